# baseline (speedup 1.0000x reference)
import jax
import jax.numpy as jnp
from jax import lax
from jax.experimental import pallas as pl
from jax.experimental.pallas import tpu as pltpu
from jax.experimental.pallas import tpu_sc as plsc

NUM_FACTORS = 32
MU = 3.5
BATCH = 16384
NC = 2
NS = 16
L = 16
NW = NC * NS
BPW = BATCH // NW
CHUNK = 128
NCHUNK = BPW // CHUNK


def _sc_body(uidx_hbm, iidx_hbm, uwt_hbm, iwt_hbm, ub_hbm, ib_hbm, out_hbm,
             uidx_v, iidx_v, ubuf_v, ibuf_v, ub_v, ib_v, out_v, sem):
    c = lax.axis_index("c")
    s = lax.axis_index("s")
    wid = s * NC + c

    pltpu.sync_copy(uidx_hbm.at[wid], uidx_v)
    pltpu.sync_copy(iidx_hbm.at[wid], iidx_v)

    copies = []
    for j in range(NCHUNK):
        dst = pl.ds(j * CHUNK, CHUNK)
        copies.append(
            pltpu.async_copy(ub_hbm.at[uidx_v.at[j]], ub_v.at[dst], sem))
        copies.append(
            pltpu.async_copy(ib_hbm.at[iidx_v.at[j]], ib_v.at[dst], sem))
    for f in range(NUM_FACTORS):
        for j in range(NCHUNK):
            dst = pl.ds(j * CHUNK, CHUNK)
            copies.append(pltpu.async_copy(
                uwt_hbm.at[f].at[uidx_v.at[j]], ubuf_v.at[f, dst], sem))
            copies.append(pltpu.async_copy(
                iwt_hbm.at[f].at[iidx_v.at[j]], ibuf_v.at[f, dst], sem))
    for cp in copies:
        cp.wait()

    def group(g, carry):
        sl = pl.ds(g * L, L)
        acc = ub_v[sl] + ib_v[sl] + MU
        for f in range(NUM_FACTORS):
            acc = acc + ubuf_v[f, sl] * ibuf_v[f, sl]
        out_v[sl] = acc
        return carry

    lax.fori_loop(0, BPW // L, group, 0)

    pltpu.sync_copy(out_v, out_hbm.at[pl.ds(wid * BPW, BPW)])


@jax.jit
def _run(uidx3, iidx3, uwt, iwt, ubf, ibf):
    mesh = plsc.VectorSubcoreMesh(core_axis_name="c", subcore_axis_name="s")
    f = pl.kernel(
        _sc_body,
        mesh=mesh,
        compiler_params=pltpu.CompilerParams(use_tc_tiling_on_sc=False),
        out_type=jax.ShapeDtypeStruct((BATCH,), jnp.float32),
        scratch_types=[
            pltpu.VMEM((NCHUNK, CHUNK), jnp.int32),
            pltpu.VMEM((NCHUNK, CHUNK), jnp.int32),
            pltpu.VMEM((NUM_FACTORS, BPW), jnp.float32),
            pltpu.VMEM((NUM_FACTORS, BPW), jnp.float32),
            pltpu.VMEM((BPW,), jnp.float32),
            pltpu.VMEM((BPW,), jnp.float32),
            pltpu.VMEM((BPW,), jnp.float32),
            pltpu.SemaphoreType.DMA,
        ],
    )
    return f(uidx3, iidx3, uwt, iwt, ubf, ibf)


def kernel(user_idx, item_idx, embed_user_w, embed_item_w, user_bias_w, item_bias_w):
    uidx3 = user_idx.reshape(NW, NCHUNK, CHUNK)
    iidx3 = item_idx.reshape(NW, NCHUNK, CHUNK)
    uwt = embed_user_w.T
    iwt = embed_item_w.T
    ubf = user_bias_w.reshape(-1)
    ibf = item_bias_w.reshape(-1)
    return _run(uidx3, iidx3, uwt, iwt, ubf, ibf)
